# speedup vs baseline: 5.3721x; 5.3721x over previous
import functools
import jax
import jax.numpy as jnp
from jax import lax
from jax.experimental import pallas as pl
from jax.experimental.pallas import tpu as pltpu

N = 10000
E = 320000
H = 4
DH = 512
NG = 64
NB = 10
RB = N // NB
F32 = jnp.float32


def _leaky(v):
    return jnp.where(v > 0, v, 0.2 * v)


def _mm_aa_body(x_ref, w_ref, asad_ref, h_ref, aa_ref):
    hb = jnp.dot(x_ref[...], w_ref[...], preferred_element_type=F32)
    h_ref[...] = hb
    aa_ref[...] = jnp.dot(hb, asad_ref[...], preferred_element_type=F32)


def mm_aa(x, W, asad):
    F = x.shape[1]
    return pl.pallas_call(
        _mm_aa_body,
        grid=(NB,),
        in_specs=[
            pl.BlockSpec((RB, F), lambda i: (i, 0)),
            pl.BlockSpec((F, H * DH), lambda i: (0, 0)),
            pl.BlockSpec((H * DH, 8), lambda i: (0, 0)),
        ],
        out_specs=[
            pl.BlockSpec((RB, H * DH), lambda i: (i, 0)),
            pl.BlockSpec((RB, 8), lambda i: (i, 0)),
        ],
        out_shape=[
            jax.ShapeDtypeStruct((N, H * DH), F32),
            jax.ShapeDtypeStruct((N, 8), F32),
        ],
    )(x, W, asad)


def _prep_body(aa_ref, gmax_ref, wself_ref):
    aa = aa_ref[...]
    asrc = aa[:, :4]
    adst = aa[:, 4:]
    gmax = jnp.max(asrc, axis=0, keepdims=True)
    c = _leaky(gmax + adst)
    wself = jnp.exp(_leaky(asrc + adst) - c)
    z4 = jnp.zeros_like(gmax)
    gmax_ref[...] = jnp.concatenate([gmax, z4], axis=1)
    wself_ref[...] = jnp.concatenate([wself, jnp.zeros_like(wself)], axis=1)


def prep(aa8):
    return pl.pallas_call(
        _prep_body,
        out_shape=[
            jax.ShapeDtypeStruct((1, 8), F32),
            jax.ShapeDtypeStruct((N, 8), F32),
        ],
    )(aa8)


def _inv_body(den_ref, wself_ref, inv_ref):
    den = den_ref[0] + den_ref[1]
    d4 = den[:, :4] + wself_ref[...][:, :4]
    inv = 1.0 / d4
    salpha = wself_ref[...][:, :4] * inv
    inv_ref[...] = jnp.concatenate([inv, salpha], axis=1)


def invden(den2, wself8):
    return pl.pallas_call(
        _inv_body,
        out_shape=jax.ShapeDtypeStruct((N, 8), F32),
    )(den2, wself8)


def _post_body(agg_ref, h_ref, inv_ref, b_ref, gat_ref, st_ref):
    pid = pl.program_id(0)
    hb = h_ref[...]
    acc = agg_ref[...]
    for hh in range(H):
        sa = inv_ref[:, pl.ds(4 + hh, 1)]
        acc = acc + hb[:, hh * DH:(hh + 1) * DH] * sa
    g = acc * (1.0 / H) + b_ref[...]
    gat_ref[...] = g
    s1 = jnp.sum(g, axis=0, keepdims=True)
    s2 = jnp.sum(g * g, axis=0, keepdims=True)
    blk = jnp.concatenate([s1, s2], axis=0)

    @pl.when(pid == 0)
    def _():
        st_ref[...] = jnp.zeros_like(st_ref)

    st_ref[...] += blk


def post(agg, h, inv8, bias):
    return pl.pallas_call(
        _post_body,
        grid=(NB,),
        in_specs=[
            pl.BlockSpec((RB, DH), lambda i: (i, 0)),
            pl.BlockSpec((RB, H * DH), lambda i: (i, 0)),
            pl.BlockSpec((RB, 8), lambda i: (i, 0)),
            pl.BlockSpec((1, DH), lambda i: (0, 0)),
        ],
        out_specs=[
            pl.BlockSpec((RB, DH), lambda i: (i, 0)),
            pl.BlockSpec((2, DH), lambda i: (0, 0)),
        ],
        out_shape=[
            jax.ShapeDtypeStruct((N, DH), F32),
            jax.ShapeDtypeStruct((2, DH), F32),
        ],
    )(agg, h, inv8, bias)


def _bnpool_body(gat_ref, st_ref, g_ref, be_ref, bt_ref, y_ref, p_ref):
    pid = pl.program_id(0)
    st = st_ref[...]
    mu = st[0:1] * (1.0 / N)
    var = st[1:2] * (1.0 / N) - mu * mu
    scale = g_ref[...] / jnp.sqrt(var + 1e-5)
    shift = be_ref[...] - mu * scale
    y = jnp.maximum(gat_ref[...] * scale + shift, 0.0)
    y_ref[...] = y
    bb = bt_ref[0]
    gi = lax.broadcasted_iota(jnp.int32, (NG, RB), 0)
    oh = (gi == bb).astype(F32)
    pb = jnp.dot(oh, y, preferred_element_type=F32)

    @pl.when(pid == 0)
    def _():
        p_ref[...] = jnp.zeros_like(p_ref)

    p_ref[...] += pb


def bnpool(gat, stats, gamma, beta, batch3):
    return pl.pallas_call(
        _bnpool_body,
        grid=(NB,),
        in_specs=[
            pl.BlockSpec((RB, DH), lambda i: (i, 0)),
            pl.BlockSpec((2, DH), lambda i: (0, 0)),
            pl.BlockSpec((1, DH), lambda i: (0, 0)),
            pl.BlockSpec((1, DH), lambda i: (0, 0)),
            pl.BlockSpec((1, 1, RB), lambda i: (i, 0, 0)),
        ],
        out_specs=[
            pl.BlockSpec((RB, DH), lambda i: (i, 0)),
            pl.BlockSpec((NG, DH), lambda i: (0, 0)),
        ],
        out_shape=[
            jax.ShapeDtypeStruct((N, DH), F32),
            jax.ShapeDtypeStruct((NG, DH), F32),
        ],
    )(gat, stats, gamma, beta, batch3)


def _mlp_body(p1_ref, p2_ref, p3_ref, w1_ref, b1_ref, w2_ref, b2_ref, o_ref):
    cat = jnp.concatenate([p1_ref[...], p2_ref[...], p3_ref[...]], axis=1)
    z = jnp.maximum(jnp.dot(cat, w1_ref[...], preferred_element_type=F32)
                    + b1_ref[...], 0.0)
    o_ref[...] = jnp.dot(z, w2_ref[...], preferred_element_type=F32) + b2_ref[...]


def mlp(p1, p2, p3, fw1, fb1, fw2, fb2):
    return pl.pallas_call(
        _mlp_body,
        out_shape=jax.ShapeDtypeStruct((NG, 10), F32),
    )(p1, p2, p3, fw1, fb1.reshape(1, -1), fw2, fb2.reshape(1, -1))


def _edges_jnp(src, dst, aa8, gmax8, wself8, h):
    asrc, adst = aa8[:, :4], aa8[:, 4:]
    gmax = gmax8[0:1, :4]
    c = _leaky(gmax + adst)
    e = _leaky(asrc[src] + adst[dst])
    w = jnp.exp(e - c[dst])
    den = jax.ops.segment_sum(w, dst, num_segments=N)
    den2 = jnp.zeros((2, N, 16), F32).at[0, :, :4].set(den)
    inv8 = invden(den2, wself8)
    alpha = w * inv8[:, :4][dst]
    hh = h.reshape(N, H, DH)
    agg = jax.ops.segment_sum((hh[src] * alpha[:, :, None]).sum(axis=1),
                              dst, num_segments=N)
    return agg, inv8


def _gat_layer(xin, src, dst, W, asad, bias, gamma, beta, batch3):
    h, aa8 = mm_aa(xin, W, asad)
    gmax8, wself8 = prep(aa8)
    agg, inv8 = _edges_jnp(src, dst, aa8, gmax8, wself8, h)
    gat, stats = post(agg, h, inv8, bias.reshape(1, DH))
    y, p = bnpool(gat, stats, gamma.reshape(1, DH), beta.reshape(1, DH), batch3)
    return y, p


def _make_asad(a_s, a_d):
    idx = jnp.arange(H * DH)
    hid = idx // DH
    As = jnp.zeros((H * DH, H), F32).at[idx, hid].set(a_s.reshape(-1))
    Ad = jnp.zeros((H * DH, H), F32).at[idx, hid].set(a_d.reshape(-1))
    return jnp.concatenate([As, Ad], axis=1)


def kernel(x, edge_index, batch, W1, as1, ad1, b1, g1, be1, W2, as2, ad2, b2,
           g2, be2, W3, as3, ad3, b3, g3, be3, fw1, fb1, fw2, fb2):
    src = edge_index[0]
    dst = edge_index[1]
    batch3 = batch.reshape(NB, 1, RB)
    y1, p1 = _gat_layer(x, src, dst, W1, _make_asad(as1, ad1), b1, g1, be1, batch3)
    y2, p2 = _gat_layer(y1, src, dst, W2, _make_asad(as2, ad2), b2, g2, be2, batch3)
    y3, p3 = _gat_layer(y2, src, dst, W3, _make_asad(as3, ad3), b3, g3, be3, batch3)
    return mlp(p1, p2, p3, fw1, fb1, fw2, fb2)


# speedup vs baseline: 12.2843x; 2.2867x over previous
import functools
import jax
import jax.numpy as jnp
from jax import lax
from jax.experimental import pallas as pl
from jax.experimental.pallas import tpu as pltpu
from jax.experimental.pallas import tpu_sc as plsc

N = 10000
E = 320000
H = 4
DH = 512
NG = 64
NB = 10
RB = N // NB
F32 = jnp.float32


def _leaky(v):
    return jnp.where(v > 0, v, 0.2 * v)


def _mm_aa_body(x_ref, w_ref, asad_ref, h_ref, aa_ref):
    hb = jnp.dot(x_ref[...], w_ref[...], preferred_element_type=F32)
    h_ref[...] = hb
    aa_ref[...] = jnp.dot(hb, asad_ref[...], preferred_element_type=F32)


def mm_aa(x, W, asad):
    F = x.shape[1]
    return pl.pallas_call(
        _mm_aa_body,
        grid=(NB,),
        in_specs=[
            pl.BlockSpec((RB, F), lambda i: (i, 0)),
            pl.BlockSpec((F, H * DH), lambda i: (0, 0)),
            pl.BlockSpec((H * DH, 128), lambda i: (0, 0)),
        ],
        out_specs=[
            pl.BlockSpec((RB, H * DH), lambda i: (i, 0)),
            pl.BlockSpec((RB, 128), lambda i: (i, 0)),
        ],
        out_shape=[
            jax.ShapeDtypeStruct((N, H * DH), F32),
            jax.ShapeDtypeStruct((N, 128), F32),
        ],
    )(x, W, asad)


def _prep_body(aa_ref, gmax_ref, wself_ref):
    aa = aa_ref[...]
    asrc = aa[:, :4]
    adst = aa[:, 4:8]
    gmax = jnp.max(asrc, axis=0, keepdims=True)
    c = _leaky(gmax + adst)
    wself = jnp.exp(_leaky(asrc + adst) - c)
    z4 = jnp.zeros_like(gmax)
    gmax_ref[...] = jnp.concatenate([gmax, z4], axis=1)
    wself_ref[...] = jnp.concatenate([wself, jnp.zeros_like(wself)], axis=1)


def prep(aa):
    return pl.pallas_call(
        _prep_body,
        out_shape=[
            jax.ShapeDtypeStruct((1, 8), F32),
            jax.ShapeDtypeStruct((N, 8), F32),
        ],
    )(aa)


def _inv_body(den_ref, wself_ref, inv_ref):
    den = den_ref[0] + den_ref[1]
    d4 = den[:N, :4] + wself_ref[...][:, :4]
    inv = 1.0 / d4
    salpha = wself_ref[...][:, :4] * inv
    inv_ref[...] = jnp.concatenate(
        [inv, salpha, jnp.zeros((N, 120), F32)], axis=1)


def invden(den2, wself8):
    return pl.pallas_call(
        _inv_body,
        out_shape=jax.ShapeDtypeStruct((N, 128), F32),
    )(den2, wself8)


def _post_body(agg_ref, h_ref, inv_ref, b_ref, gat_ref, st_ref):
    pid = pl.program_id(0)
    hb = h_ref[...]
    acc = jnp.concatenate([agg_ref[q] for q in range(4)], axis=1)
    for hh in range(H):
        sa = inv_ref[:, pl.ds(4 + hh, 1)]
        acc = acc + hb[:, hh * DH:(hh + 1) * DH] * sa
    g = acc * (1.0 / H) + b_ref[...]
    gat_ref[...] = g
    s1 = jnp.sum(g, axis=0, keepdims=True)
    s2 = jnp.sum(g * g, axis=0, keepdims=True)
    blk = jnp.concatenate([s1, s2], axis=0)

    @pl.when(pid == 0)
    def _():
        st_ref[...] = jnp.zeros_like(st_ref)

    st_ref[...] += blk


def post(agg3, h, invp, bias):
    return pl.pallas_call(
        _post_body,
        grid=(NB,),
        in_specs=[
            pl.BlockSpec((4, RB, 128), lambda i: (0, i, 0)),
            pl.BlockSpec((RB, H * DH), lambda i: (i, 0)),
            pl.BlockSpec((RB, 128), lambda i: (i, 0)),
            pl.BlockSpec((1, DH), lambda i: (0, 0)),
        ],
        out_specs=[
            pl.BlockSpec((RB, DH), lambda i: (i, 0)),
            pl.BlockSpec((2, DH), lambda i: (0, 0)),
        ],
        out_shape=[
            jax.ShapeDtypeStruct((N, DH), F32),
            jax.ShapeDtypeStruct((2, DH), F32),
        ],
    )(agg3, h, invp, bias)


def _bnpool_body(gat_ref, st_ref, g_ref, be_ref, bt_ref, y_ref, p_ref):
    pid = pl.program_id(0)
    st = st_ref[...]
    mu = st[0:1] * (1.0 / N)
    var = st[1:2] * (1.0 / N) - mu * mu
    scale = g_ref[...] / jnp.sqrt(var + 1e-5)
    shift = be_ref[...] - mu * scale
    y = jnp.maximum(gat_ref[...] * scale + shift, 0.0)
    y_ref[...] = y
    bb = bt_ref[0]
    gi = lax.broadcasted_iota(jnp.int32, (NG, RB), 0)
    oh = (gi == bb).astype(F32)
    pb = jnp.dot(oh, y, preferred_element_type=F32)

    @pl.when(pid == 0)
    def _():
        p_ref[...] = jnp.zeros_like(p_ref)

    p_ref[...] += pb


def bnpool(gat, stats, gamma, beta, batch3):
    return pl.pallas_call(
        _bnpool_body,
        grid=(NB,),
        in_specs=[
            pl.BlockSpec((RB, DH), lambda i: (i, 0)),
            pl.BlockSpec((2, DH), lambda i: (0, 0)),
            pl.BlockSpec((1, DH), lambda i: (0, 0)),
            pl.BlockSpec((1, DH), lambda i: (0, 0)),
            pl.BlockSpec((1, 1, RB), lambda i: (i, 0, 0)),
        ],
        out_specs=[
            pl.BlockSpec((RB, DH), lambda i: (i, 0)),
            pl.BlockSpec((NG, DH), lambda i: (0, 0)),
        ],
        out_shape=[
            jax.ShapeDtypeStruct((N, DH), F32),
            jax.ShapeDtypeStruct((NG, DH), F32),
        ],
    )(gat, stats, gamma, beta, batch3)


def _mlp_body(p1_ref, p2_ref, p3_ref, w1_ref, b1_ref, w2_ref, b2_ref, o_ref):
    cat = jnp.concatenate([p1_ref[...], p2_ref[...], p3_ref[...]], axis=1)
    z = jnp.maximum(jnp.dot(cat, w1_ref[...], preferred_element_type=F32)
                    + b1_ref[...], 0.0)
    o_ref[...] = jnp.dot(z, w2_ref[...], preferred_element_type=F32) + b2_ref[...]


def mlp(p1, p2, p3, fw1, fb1, fw2, fb2):
    return pl.pallas_call(
        _mlp_body,
        out_shape=jax.ShapeDtypeStruct((NG, 10), F32),
    )(p1, p2, p3, fw1, fb1.reshape(1, -1), fw2, fb2.reshape(1, -1))


EB = 64
NCH = E // EB
_SC_MESH = plsc.VectorSubcoreMesh(core_axis_name="c", subcore_axis_name="s")
NCORE = 2
NSUB = 16
NW = NCORE * NSUB
NP = 10240
ROWS_T = NP // NSUB


def _sc_edgew_body(src_hbm, dst_hbm, aa_hbm, gm_hbm, z128_hbm,
                   w3d_hbm, den2_hbm,
                   gmb, srcb, dstb, asb, adb, wb, denb, zb, spden, sem):
    cid = lax.axis_index("c")
    sid = lax.axis_index("s")
    wid = sid * NCORE + cid
    r0 = sid * ROWS_T
    pltpu.sync_copy(gm_hbm, gmb)
    pltpu.sync_copy(z128_hbm, zb)
    for z in range(ROWS_T // 128):
        pltpu.sync_copy(zb, spden.at[pl.ds(r0 + z * 128, 128)])
    pltpu.sync_copy(z128_hbm.at[pl.ds(0, EB)], denb)
    plsc.subcore_barrier()

    il = lax.iota(jnp.int32, 16)
    gmv = gmb[pl.ds(0, 16)]

    def chunk(k, carry):
        ch = k * NW + wid

        @pl.when(ch < NCH)
        def _do():
            pltpu.sync_copy(src_hbm.at[pl.ds(ch * EB, EB)], srcb)
            pltpu.sync_copy(dst_hbm.at[pl.ds(ch * EB, EB)], dstb)
            cs = pltpu.async_copy(aa_hbm.at[srcb], asb, sem)
            cd = pltpu.async_copy(aa_hbm.at[dstb], adb, sem)
            cs.wait()
            cd.wait()
            for b in range(EB // 16):
                row = il + b * 16
                for h in range(4):
                    a1 = plsc.load_gather(asb, [row, jnp.full((16,), h, jnp.int32)])
                    a2 = plsc.load_gather(adb, [row, jnp.full((16,), 4 + h, jnp.int32)])
                    e = a1 + a2
                    e = jnp.where(e > 0, e, 0.2 * e)
                    c = gmv[h] + a2
                    c = jnp.where(c > 0, c, 0.2 * c)
                    w = jnp.exp(e - c)
                    wb[h, pl.ds(b * 16, 16)] = w
                    plsc.store_scatter(denb, [row,
                                              jnp.full((16,), h, jnp.int32)], w)
            pltpu.sync_copy(wb, w3d_hbm.at[ch])
            pltpu.sync_copy(denb, spden.at[dstb], add=True)
        return carry

    lax.fori_loop(0, (NCH + NW - 1) // NW, chunk, None)
    plsc.subcore_barrier()
    for z in range(ROWS_T // 128):
        pltpu.sync_copy(spden.at[pl.ds(r0 + z * 128, 128)], zb)
        pltpu.sync_copy(zb, den2_hbm.at[cid, pl.ds(r0 + z * 128, 128)])


@functools.partial(
    pl.kernel, mesh=_SC_MESH,
    compiler_params=pltpu.CompilerParams(needs_layout_passes=False),
    out_type=[jax.ShapeDtypeStruct((NCH, 4, EB), F32),
              jax.ShapeDtypeStruct((2, NP, 128), F32)],
    scratch_types=[
        pltpu.VMEM((16,), F32),
        pltpu.VMEM((EB,), jnp.int32),
        pltpu.VMEM((EB,), jnp.int32),
        pltpu.VMEM((EB, 128), F32),
        pltpu.VMEM((EB, 128), F32),
        pltpu.VMEM((4, EB), F32),
        pltpu.VMEM((EB, 128), F32),
        pltpu.VMEM((128, 128), F32),
        pltpu.VMEM_SHARED((NP, 128), F32),
        pltpu.SemaphoreType.DMA,
    ],
)
def sc_edgew(src_hbm, dst_hbm, aa_hbm, gm_hbm, z128_hbm, w3d_hbm, den2_hbm,
             gmb, srcb, dstb, asb, adb, wb, denb, zb, spden, sem):
    _sc_edgew_body(src_hbm, dst_hbm, aa_hbm, gm_hbm, z128_hbm,
                   w3d_hbm, den2_hbm,
                   gmb, srcb, dstb, asb, adb, wb, denb, zb, spden, sem)


def _sc_alpha_body(dst_hbm, w3d_hbm, inv_hbm, al3d_hbm,
                   dstb, ib, wb, ab, sem):
    cid = lax.axis_index("c")
    sid = lax.axis_index("s")
    wid = sid * NCORE + cid
    il = lax.iota(jnp.int32, 16)

    def chunk(k, carry):
        ch = k * NW + wid

        @pl.when(ch < NCH)
        def _do():
            pltpu.sync_copy(dst_hbm.at[pl.ds(ch * EB, EB)], dstb)
            pltpu.sync_copy(w3d_hbm.at[ch], wb)
            pltpu.async_copy(inv_hbm.at[dstb], ib, sem).wait()
            for b in range(EB // 16):
                row = il + b * 16
                for h in range(4):
                    iv = plsc.load_gather(ib, [row, jnp.full((16,), h, jnp.int32)])
                    ab[h, pl.ds(b * 16, 16)] = wb[h, pl.ds(b * 16, 16)] * iv
            pltpu.sync_copy(ab, al3d_hbm.at[ch])
        return carry

    lax.fori_loop(0, (NCH + NW - 1) // NW, chunk, None)


@functools.partial(
    pl.kernel, mesh=_SC_MESH,
    compiler_params=pltpu.CompilerParams(needs_layout_passes=False),
    out_type=jax.ShapeDtypeStruct((NCH, 4, EB), F32),
    scratch_types=[
        pltpu.VMEM((EB,), jnp.int32),
        pltpu.VMEM((EB, 128), F32),
        pltpu.VMEM((4, EB), F32),
        pltpu.VMEM((4, EB), F32),
        pltpu.SemaphoreType.DMA,
    ],
)
def sc_alpha(dst_hbm, w3d_hbm, inv_hbm, al3d_hbm, dstb, ib, wb, ab, sem):
    _sc_alpha_body(dst_hbm, w3d_hbm, inv_hbm, al3d_hbm, dstb, ib, wb, ab, sem)


def _sc_agg_body(src_hbm, dst_hbm, al3d_hbm, hf_hbm, zout_hbm, agg_hbm,
                 srcb, dstb, alb, idx2, gbuf, msgb, spout, sem):
    cid = lax.axis_index("c")
    sid = lax.axis_index("s")
    r0 = sid * ROWS_T

    for cc in range(2):
        cch = cid * 2 + cc
        pltpu.sync_copy(zout_hbm, msgb)
        for z in range(ROWS_T // EB):
            pltpu.sync_copy(msgb, spout.at[pl.ds(r0 + z * EB, EB)])
        plsc.subcore_barrier()

        def chunk(k, carry):
            ch = k * NSUB + sid

            @pl.when(ch < NCH)
            def _do():
                pltpu.sync_copy(src_hbm.at[pl.ds(ch * EB, EB)], srcb)
                pltpu.sync_copy(dst_hbm.at[pl.ds(ch * EB, EB)], dstb)
                pltpu.sync_copy(al3d_hbm.at[ch], alb)
                for b in range(EB // 16):
                    sv = srcb[pl.ds(b * 16, 16)]
                    for h in range(4):
                        idx2[h, pl.ds(b * 16, 16)] = sv * 16 + (h * 4 + cch)
                cps = [pltpu.async_copy(hf_hbm.at[idx2.at[h]],
                                        gbuf.at[pl.ds(h * EB, EB)], sem)
                       for h in range(4)]
                for cp in cps:
                    cp.wait()

                def edge(j, ecarry):
                    jb = jnp.full((16,), 0, jnp.int32) + j
                    h0 = jnp.full((16,), 0, jnp.int32)
                    a0 = plsc.load_gather(alb, [h0, jb])
                    a1 = plsc.load_gather(alb, [h0 + 1, jb])
                    a2 = plsc.load_gather(alb, [h0 + 2, jb])
                    a3 = plsc.load_gather(alb, [h0 + 3, jb])
                    for r in range(8):
                        acc = (a0 * gbuf[j, pl.ds(r * 16, 16)]
                               + a1 * gbuf[EB + j, pl.ds(r * 16, 16)]
                               + a2 * gbuf[2 * EB + j, pl.ds(r * 16, 16)]
                               + a3 * gbuf[3 * EB + j, pl.ds(r * 16, 16)])
                        msgb[j, pl.ds(r * 16, 16)] = acc
                    return ecarry

                lax.fori_loop(0, EB, edge, None)
                pltpu.sync_copy(msgb, spout.at[dstb], add=True)
            return carry

        lax.fori_loop(0, (NCH + NSUB - 1) // NSUB, chunk, None)
        plsc.subcore_barrier()
        for z in range(ROWS_T // EB):
            pltpu.sync_copy(spout.at[pl.ds(r0 + z * EB, EB)], msgb)
            pltpu.sync_copy(msgb, agg_hbm.at[cch, pl.ds(r0 + z * EB, EB)])
        plsc.subcore_barrier()


sc_agg = functools.partial(
    pl.kernel, mesh=_SC_MESH,
    compiler_params=pltpu.CompilerParams(needs_layout_passes=False),
    out_type=jax.ShapeDtypeStruct((4, NP, 128), F32),
    scratch_types=[
        pltpu.VMEM((EB,), jnp.int32),
        pltpu.VMEM((EB,), jnp.int32),
        pltpu.VMEM((4, EB), F32),
        pltpu.VMEM((4, EB), jnp.int32),
        pltpu.VMEM((4 * EB, 128), F32),
        pltpu.VMEM((EB, 128), F32),
        pltpu.VMEM_SHARED((NP, 128), F32),
        pltpu.SemaphoreType.DMA,
    ],
)(_sc_agg_body)


def _edges_sc(src, dst, aa, gmax8, wself8, h):
    gm16 = jnp.pad(gmax8.reshape(-1), (0, 8))
    z128 = jnp.zeros((128, 128), F32)
    w3d, den2 = sc_edgew(src, dst, aa, gm16, z128)
    invp = invden(den2, wself8)
    al3d = sc_alpha(dst, w3d, invp)
    hf = h.reshape(N * 16, 128)
    zout = jnp.zeros((EB, 128), F32)
    agg3 = sc_agg(src, dst, al3d, hf, zout)
    return agg3, invp


def _gat_layer(xin, src, dst, W, asad, bias, gamma, beta, batch3):
    h, aa = mm_aa(xin, W, asad)
    gmax8, wself8 = prep(aa)
    agg3, invp = _edges_sc(src, dst, aa, gmax8, wself8, h)
    gat, stats = post(agg3, h, invp, bias.reshape(1, DH))
    y, p = bnpool(gat, stats, gamma.reshape(1, DH), beta.reshape(1, DH), batch3)
    return y, p


def _make_asad(a_s, a_d):
    idx = jnp.arange(H * DH)
    hid = idx // DH
    As = jnp.zeros((H * DH, H), F32).at[idx, hid].set(a_s.reshape(-1))
    Ad = jnp.zeros((H * DH, H), F32).at[idx, hid].set(a_d.reshape(-1))
    return jnp.concatenate([As, Ad, jnp.zeros((H * DH, 120), F32)], axis=1)


def kernel(x, edge_index, batch, W1, as1, ad1, b1, g1, be1, W2, as2, ad2, b2,
           g2, be2, W3, as3, ad3, b3, g3, be3, fw1, fb1, fw2, fb2):
    src = edge_index[0]
    dst = edge_index[1]
    batch3 = batch.reshape(NB, 1, RB)
    y1, p1 = _gat_layer(x, src, dst, W1, _make_asad(as1, ad1), b1, g1, be1, batch3)
    y2, p2 = _gat_layer(y1, src, dst, W2, _make_asad(as2, ad2), b2, g2, be2, batch3)
    y3, p3 = _gat_layer(y2, src, dst, W3, _make_asad(as3, ad3), b3, g3, be3, batch3)
    return mlp(p1, p2, p3, fw1, fb1, fw2, fb2)


# speedup vs baseline: 13.3680x; 1.0882x over previous
import functools
import jax
import jax.numpy as jnp
from jax import lax
from jax.experimental import pallas as pl
from jax.experimental.pallas import tpu as pltpu
from jax.experimental.pallas import tpu_sc as plsc

N = 10000
E = 320000
H = 4
DH = 512
NG = 64
NB = 10
RB = N // NB
F32 = jnp.float32


def _leaky(v):
    return jnp.where(v > 0, v, 0.2 * v)


def _mm_aa_body(x_ref, w_ref, asad_ref, h_ref, aa_ref):
    hb = jnp.dot(x_ref[...], w_ref[...], preferred_element_type=F32)
    h_ref[...] = hb
    aa_ref[...] = jnp.dot(hb, asad_ref[...], preferred_element_type=F32)


def mm_aa(x, W, asad):
    F = x.shape[1]
    return pl.pallas_call(
        _mm_aa_body,
        grid=(NB,),
        in_specs=[
            pl.BlockSpec((RB, F), lambda i: (i, 0)),
            pl.BlockSpec((F, H * DH), lambda i: (0, 0)),
            pl.BlockSpec((H * DH, 128), lambda i: (0, 0)),
        ],
        out_specs=[
            pl.BlockSpec((RB, H * DH), lambda i: (i, 0)),
            pl.BlockSpec((RB, 128), lambda i: (i, 0)),
        ],
        out_shape=[
            jax.ShapeDtypeStruct((N, H * DH), F32),
            jax.ShapeDtypeStruct((N, 128), F32),
        ],
    )(x, W, asad)


def _prep_body(aa_ref, gmax_ref, wself_ref):
    aa = aa_ref[...]
    asrc = aa[:, :4]
    adst = aa[:, 4:8]
    gmax = jnp.max(asrc, axis=0, keepdims=True)
    c = _leaky(gmax + adst)
    wself = jnp.exp(_leaky(asrc + adst) - c)
    z4 = jnp.zeros_like(gmax)
    gmax_ref[...] = jnp.concatenate([gmax, z4], axis=1)
    wself_ref[...] = jnp.concatenate([wself, jnp.zeros_like(wself)], axis=1)


def prep(aa):
    return pl.pallas_call(
        _prep_body,
        out_shape=[
            jax.ShapeDtypeStruct((1, 8), F32),
            jax.ShapeDtypeStruct((N, 8), F32),
        ],
    )(aa)


def _inv_body(den_ref, wself_ref, inv_ref):
    den = den_ref[0] + den_ref[1]
    d4 = den[:N, :4] + wself_ref[...][:, :4]
    inv = 1.0 / d4
    salpha = wself_ref[...][:, :4] * inv
    inv_ref[...] = jnp.concatenate(
        [inv, salpha, jnp.zeros((N, 120), F32)], axis=1)


def invden(den2, wself8):
    return pl.pallas_call(
        _inv_body,
        out_shape=jax.ShapeDtypeStruct((N, 128), F32),
    )(den2, wself8)


def _post_body(agg_ref, h_ref, inv_ref, b_ref, gat_ref, st_ref):
    pid = pl.program_id(0)
    hb = h_ref[...]
    acc = jnp.concatenate([agg_ref[q] for q in range(4)], axis=1)
    for hh in range(H):
        sa = inv_ref[:, pl.ds(4 + hh, 1)]
        acc = acc + hb[:, hh * DH:(hh + 1) * DH] * sa
    g = acc * (1.0 / H) + b_ref[...]
    gat_ref[...] = g
    s1 = jnp.sum(g, axis=0, keepdims=True)
    s2 = jnp.sum(g * g, axis=0, keepdims=True)
    blk = jnp.concatenate([s1, s2], axis=0)

    @pl.when(pid == 0)
    def _():
        st_ref[...] = jnp.zeros_like(st_ref)

    st_ref[...] += blk


def post(agg3, h, invp, bias):
    return pl.pallas_call(
        _post_body,
        grid=(NB,),
        in_specs=[
            pl.BlockSpec((4, RB, 128), lambda i: (0, i, 0)),
            pl.BlockSpec((RB, H * DH), lambda i: (i, 0)),
            pl.BlockSpec((RB, 128), lambda i: (i, 0)),
            pl.BlockSpec((1, DH), lambda i: (0, 0)),
        ],
        out_specs=[
            pl.BlockSpec((RB, DH), lambda i: (i, 0)),
            pl.BlockSpec((2, DH), lambda i: (0, 0)),
        ],
        out_shape=[
            jax.ShapeDtypeStruct((N, DH), F32),
            jax.ShapeDtypeStruct((2, DH), F32),
        ],
    )(agg3, h, invp, bias)


def _bnpool_body(gat_ref, st_ref, g_ref, be_ref, bt_ref, y_ref, p_ref):
    pid = pl.program_id(0)
    st = st_ref[...]
    mu = st[0:1] * (1.0 / N)
    var = st[1:2] * (1.0 / N) - mu * mu
    scale = g_ref[...] / jnp.sqrt(var + 1e-5)
    shift = be_ref[...] - mu * scale
    y = jnp.maximum(gat_ref[...] * scale + shift, 0.0)
    y_ref[...] = y
    bb = bt_ref[0]
    gi = lax.broadcasted_iota(jnp.int32, (NG, RB), 0)
    oh = (gi == bb).astype(F32)
    pb = jnp.dot(oh, y, preferred_element_type=F32)

    @pl.when(pid == 0)
    def _():
        p_ref[...] = jnp.zeros_like(p_ref)

    p_ref[...] += pb


def bnpool(gat, stats, gamma, beta, batch3):
    return pl.pallas_call(
        _bnpool_body,
        grid=(NB,),
        in_specs=[
            pl.BlockSpec((RB, DH), lambda i: (i, 0)),
            pl.BlockSpec((2, DH), lambda i: (0, 0)),
            pl.BlockSpec((1, DH), lambda i: (0, 0)),
            pl.BlockSpec((1, DH), lambda i: (0, 0)),
            pl.BlockSpec((1, 1, RB), lambda i: (i, 0, 0)),
        ],
        out_specs=[
            pl.BlockSpec((RB, DH), lambda i: (i, 0)),
            pl.BlockSpec((NG, DH), lambda i: (0, 0)),
        ],
        out_shape=[
            jax.ShapeDtypeStruct((N, DH), F32),
            jax.ShapeDtypeStruct((NG, DH), F32),
        ],
    )(gat, stats, gamma, beta, batch3)


def _mlp_body(p1_ref, p2_ref, p3_ref, w1_ref, b1_ref, w2_ref, b2_ref, o_ref):
    cat = jnp.concatenate([p1_ref[...], p2_ref[...], p3_ref[...]], axis=1)
    z = jnp.maximum(jnp.dot(cat, w1_ref[...], preferred_element_type=F32)
                    + b1_ref[...], 0.0)
    o_ref[...] = jnp.dot(z, w2_ref[...], preferred_element_type=F32) + b2_ref[...]


def mlp(p1, p2, p3, fw1, fb1, fw2, fb2):
    return pl.pallas_call(
        _mlp_body,
        out_shape=jax.ShapeDtypeStruct((NG, 10), F32),
    )(p1, p2, p3, fw1, fb1.reshape(1, -1), fw2, fb2.reshape(1, -1))


EB = 64
NCH = E // EB
_SC_MESH = plsc.VectorSubcoreMesh(core_axis_name="c", subcore_axis_name="s")
NCORE = 2
NSUB = 16
NW = NCORE * NSUB
NP = 10240
ROWS_T = NP // NSUB


def _sc_edgew_body(src_hbm, dst_hbm, aa_hbm, gm_hbm, z128_hbm,
                   w3d_hbm, den2_hbm,
                   gmb, srcb, dstb, asb, adb, wb, denb, zb, spden, sem):
    cid = lax.axis_index("c")
    sid = lax.axis_index("s")
    wid = sid * NCORE + cid
    r0 = sid * ROWS_T
    pltpu.sync_copy(gm_hbm, gmb)
    pltpu.sync_copy(z128_hbm, zb)
    for z in range(ROWS_T // 128):
        pltpu.sync_copy(zb, spden.at[pl.ds(r0 + z * 128, 128)])
    pltpu.sync_copy(z128_hbm.at[pl.ds(0, EB)], denb)
    plsc.subcore_barrier()

    il = lax.iota(jnp.int32, 16)
    gmv = gmb[pl.ds(0, 16)]

    def chunk(k, carry):
        ch = k * NW + wid

        @pl.when(ch < NCH)
        def _do():
            pltpu.sync_copy(src_hbm.at[pl.ds(ch * EB, EB)], srcb)
            pltpu.sync_copy(dst_hbm.at[pl.ds(ch * EB, EB)], dstb)
            cs = pltpu.async_copy(aa_hbm.at[srcb], asb, sem)
            cd = pltpu.async_copy(aa_hbm.at[dstb], adb, sem)
            cs.wait()
            cd.wait()
            for b in range(EB // 16):
                row = il + b * 16
                for h in range(4):
                    a1 = plsc.load_gather(asb, [row, jnp.full((16,), h, jnp.int32)])
                    a2 = plsc.load_gather(adb, [row, jnp.full((16,), 4 + h, jnp.int32)])
                    e = a1 + a2
                    e = jnp.where(e > 0, e, 0.2 * e)
                    c = gmv[h] + a2
                    c = jnp.where(c > 0, c, 0.2 * c)
                    w = jnp.exp(e - c)
                    wb[h, pl.ds(b * 16, 16)] = w
                    plsc.store_scatter(denb, [row,
                                              jnp.full((16,), h, jnp.int32)], w)
            pltpu.sync_copy(wb, w3d_hbm.at[ch])
            pltpu.sync_copy(denb, spden.at[dstb], add=True)
        return carry

    lax.fori_loop(0, (NCH + NW - 1) // NW, chunk, None)
    plsc.subcore_barrier()
    for z in range(ROWS_T // 128):
        pltpu.sync_copy(spden.at[pl.ds(r0 + z * 128, 128)], zb)
        pltpu.sync_copy(zb, den2_hbm.at[cid, pl.ds(r0 + z * 128, 128)])


@functools.partial(
    pl.kernel, mesh=_SC_MESH,
    compiler_params=pltpu.CompilerParams(needs_layout_passes=False),
    out_type=[jax.ShapeDtypeStruct((NCH, 4, EB), F32),
              jax.ShapeDtypeStruct((2, NP, 128), F32)],
    scratch_types=[
        pltpu.VMEM((16,), F32),
        pltpu.VMEM((EB,), jnp.int32),
        pltpu.VMEM((EB,), jnp.int32),
        pltpu.VMEM((EB, 128), F32),
        pltpu.VMEM((EB, 128), F32),
        pltpu.VMEM((4, EB), F32),
        pltpu.VMEM((EB, 128), F32),
        pltpu.VMEM((128, 128), F32),
        pltpu.VMEM_SHARED((NP, 128), F32),
        pltpu.SemaphoreType.DMA,
    ],
)
def sc_edgew(src_hbm, dst_hbm, aa_hbm, gm_hbm, z128_hbm, w3d_hbm, den2_hbm,
             gmb, srcb, dstb, asb, adb, wb, denb, zb, spden, sem):
    _sc_edgew_body(src_hbm, dst_hbm, aa_hbm, gm_hbm, z128_hbm,
                   w3d_hbm, den2_hbm,
                   gmb, srcb, dstb, asb, adb, wb, denb, zb, spden, sem)


def _sc_alpha_body(dst_hbm, w3d_hbm, inv_hbm, al3d_hbm,
                   dstb, ib, wb, ab, sem):
    cid = lax.axis_index("c")
    sid = lax.axis_index("s")
    wid = sid * NCORE + cid
    il = lax.iota(jnp.int32, 16)

    def chunk(k, carry):
        ch = k * NW + wid

        @pl.when(ch < NCH)
        def _do():
            pltpu.sync_copy(dst_hbm.at[pl.ds(ch * EB, EB)], dstb)
            pltpu.sync_copy(w3d_hbm.at[ch], wb)
            pltpu.async_copy(inv_hbm.at[dstb], ib, sem).wait()
            for b in range(EB // 16):
                row = il + b * 16
                for h in range(4):
                    iv = plsc.load_gather(ib, [row, jnp.full((16,), h, jnp.int32)])
                    ab[h, pl.ds(b * 16, 16)] = wb[h, pl.ds(b * 16, 16)] * iv
            pltpu.sync_copy(ab, al3d_hbm.at[ch])
        return carry

    lax.fori_loop(0, (NCH + NW - 1) // NW, chunk, None)


@functools.partial(
    pl.kernel, mesh=_SC_MESH,
    compiler_params=pltpu.CompilerParams(needs_layout_passes=False),
    out_type=jax.ShapeDtypeStruct((NCH, 4, EB), F32),
    scratch_types=[
        pltpu.VMEM((EB,), jnp.int32),
        pltpu.VMEM((EB, 128), F32),
        pltpu.VMEM((4, EB), F32),
        pltpu.VMEM((4, EB), F32),
        pltpu.SemaphoreType.DMA,
    ],
)
def sc_alpha(dst_hbm, w3d_hbm, inv_hbm, al3d_hbm, dstb, ib, wb, ab, sem):
    _sc_alpha_body(dst_hbm, w3d_hbm, inv_hbm, al3d_hbm, dstb, ib, wb, ab, sem)


AB = 32
NCH32 = E // AB


def _sc_agg_body(src_hbm, dst_hbm, al3d_hbm, hf_hbm, zout_hbm, agg_hbm,
                 srcb, dstb, alb, idx2, gbuf, msgb, spout,
                 semp0, semp1, semg0, semg1):
    cid = lax.axis_index("c")
    sid = lax.axis_index("s")
    r0 = sid * ROWS_T
    half = jnp.mod(sid, 2) * AB
    il = lax.iota(jnp.int32, 16)
    KT = NCH32 // NSUB
    sems = (semp0, semp1)
    semg = (semg0, semg1)

    def fire_reads(p, k, sem):
        ch = k * NSUB + sid
        c1 = pltpu.async_copy(src_hbm.at[pl.ds(ch * AB, AB)], srcb.at[p], sem)
        c2 = pltpu.async_copy(dst_hbm.at[pl.ds(ch * AB, AB)], dstb.at[p], sem)
        c3 = pltpu.async_copy(al3d_hbm.at[(ch * AB) // EB], alb.at[p], sem)
        return c1, c2, c3

    def wait_reads(p, sem):
        for ref in (srcb.at[p], dstb.at[p]):
            pltpu.make_async_copy(src_hbm.at[pl.ds(0, AB)], ref, sem).wait()
        pltpu.make_async_copy(al3d_hbm.at[0], alb.at[p], sem).wait()

    def prep_and_gather(p, cch):
        for b in range(AB // 16):
            sv = srcb[p, pl.ds(b * 16, 16)]
            for h in range(4):
                idx2[p, h, pl.ds(b * 16, 16)] = sv * 16 + (h * 4 + cch)
        return [pltpu.async_copy(hf_hbm.at[idx2.at[p, h]],
                                 gbuf.at[p, pl.ds(h * AB, AB)], semg[p])
                for h in range(4)]

    def wait_gathers(p):
        for h in range(4):
            pltpu.make_async_copy(hf_hbm.at[idx2.at[p, h]],
                                  gbuf.at[p, pl.ds(h * AB, AB)], semg[p]).wait()

    def compute_scatter(p):
        def edge(j, ecarry):
            jb = jnp.full((16,), 0, jnp.int32) + j + half
            h0 = jnp.full((16,), 0, jnp.int32)
            a0 = plsc.load_gather(alb.at[p], [h0, jb])
            a1 = plsc.load_gather(alb.at[p], [h0 + 1, jb])
            a2 = plsc.load_gather(alb.at[p], [h0 + 2, jb])
            a3 = plsc.load_gather(alb.at[p], [h0 + 3, jb])
            for r in range(8):
                acc = (a0 * gbuf[p, j, pl.ds(r * 16, 16)]
                       + a1 * gbuf[p, AB + j, pl.ds(r * 16, 16)]
                       + a2 * gbuf[p, 2 * AB + j, pl.ds(r * 16, 16)]
                       + a3 * gbuf[p, 3 * AB + j, pl.ds(r * 16, 16)])
                msgb[p, j, pl.ds(r * 16, 16)] = acc
            return ecarry

        lax.fori_loop(0, AB, edge, None)
        pltpu.sync_copy(msgb.at[p], spout.at[dstb.at[p]], add=True)

    for cc in range(2):
        cch = cid * 2 + cc
        pltpu.sync_copy(zout_hbm, msgb.at[0])
        for z in range(ROWS_T // AB):
            pltpu.sync_copy(msgb.at[0], spout.at[pl.ds(r0 + z * AB, AB)])
        plsc.subcore_barrier()

        for c in fire_reads(0, 0, semp0):
            pass
        wait_reads(0, semp0)
        g0 = prep_and_gather(0, cch)
        fire_reads(1, 1, semp1)

        def pair(kk, carry):
            k0 = 2 * kk
            k1 = 2 * kk + 1

            @pl.when(k1 < KT)
            def _a():
                wait_reads(1, semp1)
                prep_and_gather(1, cch)

            wait_gathers(0)
            compute_scatter(0)

            @pl.when(k0 + 2 < KT)
            def _b():
                fire_reads(0, k0 + 2, semp0)
                wait_reads(0, semp0)
                prep_and_gather(0, cch)

            @pl.when(k1 < KT)
            def _c():
                wait_gathers(1)
                compute_scatter(1)

            @pl.when(k1 + 2 < KT)
            def _d():
                fire_reads(1, k1 + 2, semp1)
            return carry

        lax.fori_loop(0, (KT + 1) // 2, pair, None)
        plsc.subcore_barrier()
        for z in range(ROWS_T // AB):
            pltpu.sync_copy(spout.at[pl.ds(r0 + z * AB, AB)], msgb.at[0])
            pltpu.sync_copy(msgb.at[0], agg_hbm.at[cch, pl.ds(r0 + z * AB, AB)])
        plsc.subcore_barrier()


sc_agg = functools.partial(
    pl.kernel, mesh=_SC_MESH,
    compiler_params=pltpu.CompilerParams(needs_layout_passes=False),
    out_type=jax.ShapeDtypeStruct((4, NP, 128), F32),
    scratch_types=[
        pltpu.VMEM((2, AB), jnp.int32),
        pltpu.VMEM((2, AB), jnp.int32),
        pltpu.VMEM((2, 4, EB), F32),
        pltpu.VMEM((2, 4, AB), jnp.int32),
        pltpu.VMEM((2, 4 * AB, 128), F32),
        pltpu.VMEM((2, AB, 128), F32),
        pltpu.VMEM_SHARED((NP, 128), F32),
        pltpu.SemaphoreType.DMA,
        pltpu.SemaphoreType.DMA,
        pltpu.SemaphoreType.DMA,
        pltpu.SemaphoreType.DMA,
    ],
)(_sc_agg_body)


def _edges_sc(src, dst, aa, gmax8, wself8, h):
    gm16 = jnp.pad(gmax8.reshape(-1), (0, 8))
    z128 = jnp.zeros((128, 128), F32)
    w3d, den2 = sc_edgew(src, dst, aa, gm16, z128)
    invp = invden(den2, wself8)
    al3d = sc_alpha(dst, w3d, invp)
    hf = h.reshape(N * 16, 128)
    zout = jnp.zeros((AB, 128), F32)
    agg3 = sc_agg(src, dst, al3d, hf, zout)
    return agg3, invp


def _gat_layer(xin, src, dst, W, asad, bias, gamma, beta, batch3):
    h, aa = mm_aa(xin, W, asad)
    gmax8, wself8 = prep(aa)
    agg3, invp = _edges_sc(src, dst, aa, gmax8, wself8, h)
    gat, stats = post(agg3, h, invp, bias.reshape(1, DH))
    y, p = bnpool(gat, stats, gamma.reshape(1, DH), beta.reshape(1, DH), batch3)
    return y, p


def _make_asad(a_s, a_d):
    idx = jnp.arange(H * DH)
    hid = idx // DH
    As = jnp.zeros((H * DH, H), F32).at[idx, hid].set(a_s.reshape(-1))
    Ad = jnp.zeros((H * DH, H), F32).at[idx, hid].set(a_d.reshape(-1))
    return jnp.concatenate([As, Ad, jnp.zeros((H * DH, 120), F32)], axis=1)


def kernel(x, edge_index, batch, W1, as1, ad1, b1, g1, be1, W2, as2, ad2, b2,
           g2, be2, W3, as3, ad3, b3, g3, be3, fw1, fb1, fw2, fb2):
    src = edge_index[0]
    dst = edge_index[1]
    batch3 = batch.reshape(NB, 1, RB)
    y1, p1 = _gat_layer(x, src, dst, W1, _make_asad(as1, ad1), b1, g1, be1, batch3)
    y2, p2 = _gat_layer(y1, src, dst, W2, _make_asad(as2, ad2), b2, g2, be2, batch3)
    y3, p3 = _gat_layer(y2, src, dst, W3, _make_asad(as3, ad3), b3, g3, be3, batch3)
    return mlp(p1, p2, p3, fw1, fb1, fw2, fb2)


# speedup vs baseline: 14.1286x; 1.0569x over previous
import functools
import jax
import jax.numpy as jnp
from jax import lax
from jax.experimental import pallas as pl
from jax.experimental.pallas import tpu as pltpu
from jax.experimental.pallas import tpu_sc as plsc

N = 10000
E = 320000
H = 4
DH = 512
NG = 64
NB = 10
RB = N // NB
F32 = jnp.float32


def _leaky(v):
    return jnp.where(v > 0, v, 0.2 * v)


def _mm_aa_body(x_ref, w_ref, asad_ref, h_ref, aa_ref):
    hb = jnp.dot(x_ref[...], w_ref[...], preferred_element_type=F32)
    h_ref[...] = hb
    aa_ref[...] = jnp.dot(hb, asad_ref[...], preferred_element_type=F32)


def mm_aa(x, W, asad):
    F = x.shape[1]
    return pl.pallas_call(
        _mm_aa_body,
        grid=(NB,),
        in_specs=[
            pl.BlockSpec((RB, F), lambda i: (i, 0)),
            pl.BlockSpec((F, H * DH), lambda i: (0, 0)),
            pl.BlockSpec((H * DH, 128), lambda i: (0, 0)),
        ],
        out_specs=[
            pl.BlockSpec((RB, H * DH), lambda i: (i, 0)),
            pl.BlockSpec((RB, 128), lambda i: (i, 0)),
        ],
        out_shape=[
            jax.ShapeDtypeStruct((N, H * DH), F32),
            jax.ShapeDtypeStruct((N, 128), F32),
        ],
    )(x, W, asad)


def _prep_body(aa_ref, gmax_ref, wself_ref):
    aa = aa_ref[...]
    asrc = aa[:, :4]
    adst = aa[:, 4:8]
    gmax = jnp.max(asrc, axis=0, keepdims=True)
    c = _leaky(gmax + adst)
    wself = jnp.exp(_leaky(asrc + adst) - c)
    z4 = jnp.zeros_like(gmax)
    gmax_ref[...] = jnp.concatenate([gmax, z4], axis=1)
    wself_ref[...] = jnp.concatenate([wself, jnp.zeros_like(wself)], axis=1)


def prep(aa):
    return pl.pallas_call(
        _prep_body,
        out_shape=[
            jax.ShapeDtypeStruct((1, 8), F32),
            jax.ShapeDtypeStruct((N, 8), F32),
        ],
    )(aa)


def _inv_body(den_ref, wself_ref, inv_ref):
    den = den_ref[0] + den_ref[1]
    d4 = den[:N, :4] + wself_ref[...][:, :4]
    inv = 1.0 / d4
    salpha = wself_ref[...][:, :4] * inv
    inv_ref[...] = jnp.concatenate(
        [inv, salpha, jnp.zeros((N, 120), F32)], axis=1)


def invden(den2, wself8):
    return pl.pallas_call(
        _inv_body,
        out_shape=jax.ShapeDtypeStruct((N, 128), F32),
    )(den2, wself8)


def _post_body(agg_ref, h_ref, inv_ref, b_ref, gat_ref, st_ref):
    pid = pl.program_id(0)
    hb = h_ref[...]
    acc = jnp.concatenate([agg_ref[q] for q in range(4)], axis=1)
    for hh in range(H):
        sa = inv_ref[:, pl.ds(4 + hh, 1)]
        acc = acc + hb[:, hh * DH:(hh + 1) * DH] * sa
    g = acc * (1.0 / H) + b_ref[...]
    gat_ref[...] = g
    s1 = jnp.sum(g, axis=0, keepdims=True)
    s2 = jnp.sum(g * g, axis=0, keepdims=True)
    blk = jnp.concatenate([s1, s2], axis=0)

    @pl.when(pid == 0)
    def _():
        st_ref[...] = jnp.zeros_like(st_ref)

    st_ref[...] += blk


def post(agg3, h, invp, bias):
    return pl.pallas_call(
        _post_body,
        grid=(NB,),
        in_specs=[
            pl.BlockSpec((4, RB, 128), lambda i: (0, i, 0)),
            pl.BlockSpec((RB, H * DH), lambda i: (i, 0)),
            pl.BlockSpec((RB, 128), lambda i: (i, 0)),
            pl.BlockSpec((1, DH), lambda i: (0, 0)),
        ],
        out_specs=[
            pl.BlockSpec((RB, DH), lambda i: (i, 0)),
            pl.BlockSpec((2, DH), lambda i: (0, 0)),
        ],
        out_shape=[
            jax.ShapeDtypeStruct((N, DH), F32),
            jax.ShapeDtypeStruct((2, DH), F32),
        ],
    )(agg3, h, invp, bias)


def _bnpool_body(gat_ref, st_ref, g_ref, be_ref, bt_ref, y_ref, p_ref):
    pid = pl.program_id(0)
    st = st_ref[...]
    mu = st[0:1] * (1.0 / N)
    var = st[1:2] * (1.0 / N) - mu * mu
    scale = g_ref[...] / jnp.sqrt(var + 1e-5)
    shift = be_ref[...] - mu * scale
    y = jnp.maximum(gat_ref[...] * scale + shift, 0.0)
    y_ref[...] = y
    bb = bt_ref[0]
    gi = lax.broadcasted_iota(jnp.int32, (NG, RB), 0)
    oh = (gi == bb).astype(F32)
    pb = jnp.dot(oh, y, preferred_element_type=F32)

    @pl.when(pid == 0)
    def _():
        p_ref[...] = jnp.zeros_like(p_ref)

    p_ref[...] += pb


def bnpool(gat, stats, gamma, beta, batch3):
    return pl.pallas_call(
        _bnpool_body,
        grid=(NB,),
        in_specs=[
            pl.BlockSpec((RB, DH), lambda i: (i, 0)),
            pl.BlockSpec((2, DH), lambda i: (0, 0)),
            pl.BlockSpec((1, DH), lambda i: (0, 0)),
            pl.BlockSpec((1, DH), lambda i: (0, 0)),
            pl.BlockSpec((1, 1, RB), lambda i: (i, 0, 0)),
        ],
        out_specs=[
            pl.BlockSpec((RB, DH), lambda i: (i, 0)),
            pl.BlockSpec((NG, DH), lambda i: (0, 0)),
        ],
        out_shape=[
            jax.ShapeDtypeStruct((N, DH), F32),
            jax.ShapeDtypeStruct((NG, DH), F32),
        ],
    )(gat, stats, gamma, beta, batch3)


def _mlp_body(p1_ref, p2_ref, p3_ref, w1_ref, b1_ref, w2_ref, b2_ref, o_ref):
    cat = jnp.concatenate([p1_ref[...], p2_ref[...], p3_ref[...]], axis=1)
    z = jnp.maximum(jnp.dot(cat, w1_ref[...], preferred_element_type=F32)
                    + b1_ref[...], 0.0)
    o_ref[...] = jnp.dot(z, w2_ref[...], preferred_element_type=F32) + b2_ref[...]


def mlp(p1, p2, p3, fw1, fb1, fw2, fb2):
    return pl.pallas_call(
        _mlp_body,
        out_shape=jax.ShapeDtypeStruct((NG, 10), F32),
    )(p1, p2, p3, fw1, fb1.reshape(1, -1), fw2, fb2.reshape(1, -1))


EB = 64
NCH = E // EB
_SC_MESH = plsc.VectorSubcoreMesh(core_axis_name="c", subcore_axis_name="s")
NCORE = 2
NSUB = 16
NW = NCORE * NSUB
NP = 10240
ROWS_T = NP // NSUB


def _sc_edgew_body(src_hbm, dst_hbm, aa_hbm, gm_hbm, z128_hbm,
                   w3d_hbm, den2_hbm,
                   gmb, srcb, dstb, asb, adb, wb, denb, zb, spden, sem):
    cid = lax.axis_index("c")
    sid = lax.axis_index("s")
    wid = sid * NCORE + cid
    r0 = sid * ROWS_T
    pltpu.sync_copy(gm_hbm, gmb)
    pltpu.sync_copy(z128_hbm, zb)
    for z in range(ROWS_T // 128):
        pltpu.sync_copy(zb, spden.at[pl.ds(r0 + z * 128, 128)])
    pltpu.sync_copy(z128_hbm.at[pl.ds(0, EB)], denb)
    plsc.subcore_barrier()

    il = lax.iota(jnp.int32, 16)
    gmv = gmb[pl.ds(0, 16)]

    def chunk(k, carry):
        ch = k * NW + wid

        @pl.when(ch < NCH)
        def _do():
            pltpu.sync_copy(src_hbm.at[pl.ds(ch * EB, EB)], srcb)
            pltpu.sync_copy(dst_hbm.at[pl.ds(ch * EB, EB)], dstb)
            cs = pltpu.async_copy(aa_hbm.at[srcb], asb, sem)
            cd = pltpu.async_copy(aa_hbm.at[dstb], adb, sem)
            cs.wait()
            cd.wait()
            for b in range(EB // 16):
                row = il + b * 16
                for h in range(4):
                    a1 = plsc.load_gather(asb, [row, jnp.full((16,), h, jnp.int32)])
                    a2 = plsc.load_gather(adb, [row, jnp.full((16,), 4 + h, jnp.int32)])
                    e = a1 + a2
                    e = jnp.where(e > 0, e, 0.2 * e)
                    c = gmv[h] + a2
                    c = jnp.where(c > 0, c, 0.2 * c)
                    w = jnp.exp(e - c)
                    wb[h, pl.ds(b * 16, 16)] = w
                    plsc.store_scatter(denb, [row,
                                              jnp.full((16,), h, jnp.int32)], w)
            pltpu.sync_copy(wb, w3d_hbm.at[ch])
            pltpu.sync_copy(denb, spden.at[dstb], add=True)
        return carry

    lax.fori_loop(0, (NCH + NW - 1) // NW, chunk, None)
    plsc.subcore_barrier()
    for z in range(ROWS_T // 128):
        pltpu.sync_copy(spden.at[pl.ds(r0 + z * 128, 128)], zb)
        pltpu.sync_copy(zb, den2_hbm.at[cid, pl.ds(r0 + z * 128, 128)])


@functools.partial(
    pl.kernel, mesh=_SC_MESH,
    compiler_params=pltpu.CompilerParams(needs_layout_passes=False),
    out_type=[jax.ShapeDtypeStruct((NCH, 4, EB), F32),
              jax.ShapeDtypeStruct((2, NP, 128), F32)],
    scratch_types=[
        pltpu.VMEM((16,), F32),
        pltpu.VMEM((EB,), jnp.int32),
        pltpu.VMEM((EB,), jnp.int32),
        pltpu.VMEM((EB, 128), F32),
        pltpu.VMEM((EB, 128), F32),
        pltpu.VMEM((4, EB), F32),
        pltpu.VMEM((EB, 128), F32),
        pltpu.VMEM((128, 128), F32),
        pltpu.VMEM_SHARED((NP, 128), F32),
        pltpu.SemaphoreType.DMA,
    ],
)
def sc_edgew(src_hbm, dst_hbm, aa_hbm, gm_hbm, z128_hbm, w3d_hbm, den2_hbm,
             gmb, srcb, dstb, asb, adb, wb, denb, zb, spden, sem):
    _sc_edgew_body(src_hbm, dst_hbm, aa_hbm, gm_hbm, z128_hbm,
                   w3d_hbm, den2_hbm,
                   gmb, srcb, dstb, asb, adb, wb, denb, zb, spden, sem)


def _sc_alpha_body(dst_hbm, w3d_hbm, inv_hbm, al3d_hbm,
                   dstb, ib, wb, ab, sem):
    cid = lax.axis_index("c")
    sid = lax.axis_index("s")
    wid = sid * NCORE + cid
    il = lax.iota(jnp.int32, 16)

    def chunk(k, carry):
        ch = k * NW + wid

        @pl.when(ch < NCH)
        def _do():
            pltpu.sync_copy(dst_hbm.at[pl.ds(ch * EB, EB)], dstb)
            pltpu.sync_copy(w3d_hbm.at[ch], wb)
            pltpu.async_copy(inv_hbm.at[dstb], ib, sem).wait()
            for b in range(EB // 16):
                row = il + b * 16
                for h in range(4):
                    iv = plsc.load_gather(ib, [row, jnp.full((16,), h, jnp.int32)])
                    ab[h, pl.ds(b * 16, 16)] = wb[h, pl.ds(b * 16, 16)] * iv
            pltpu.sync_copy(ab, al3d_hbm.at[ch])
        return carry

    lax.fori_loop(0, (NCH + NW - 1) // NW, chunk, None)


@functools.partial(
    pl.kernel, mesh=_SC_MESH,
    compiler_params=pltpu.CompilerParams(needs_layout_passes=False),
    out_type=jax.ShapeDtypeStruct((NCH, 4, EB), F32),
    scratch_types=[
        pltpu.VMEM((EB,), jnp.int32),
        pltpu.VMEM((EB, 128), F32),
        pltpu.VMEM((4, EB), F32),
        pltpu.VMEM((4, EB), F32),
        pltpu.SemaphoreType.DMA,
    ],
)
def sc_alpha(dst_hbm, w3d_hbm, inv_hbm, al3d_hbm, dstb, ib, wb, ab, sem):
    _sc_alpha_body(dst_hbm, w3d_hbm, inv_hbm, al3d_hbm, dstb, ib, wb, ab, sem)


AB = 32
NCH32 = E // AB


def _sc_agg_body(src_hbm, dst_hbm, al3d_hbm, hf_hbm, zout_hbm, agg_hbm,
                 srcb, dstb, sdst, alb, idx2, gbuf, msgb, spout,
                 semp0, semp1, semg0, semg1, sems0, sems1):
    cid = lax.axis_index("c")
    sid = lax.axis_index("s")
    r0 = sid * ROWS_T
    half = jnp.mod(sid, 2) * AB
    il = lax.iota(jnp.int32, 16)
    KT = NCH32 // NSUB
    sems = (semp0, semp1)
    semg = (semg0, semg1)
    semsc = (sems0, sems1)

    def fire_reads(p, k, sem):
        ch = k * NSUB + sid
        c1 = pltpu.async_copy(src_hbm.at[pl.ds(ch * AB, AB)], srcb.at[p], sem)
        c2 = pltpu.async_copy(dst_hbm.at[pl.ds(ch * AB, AB)], dstb.at[p], sem)
        c3 = pltpu.async_copy(al3d_hbm.at[(ch * AB) // EB], alb.at[p], sem)
        return c1, c2, c3

    def wait_reads(p, sem):
        for ref in (srcb.at[p], dstb.at[p]):
            pltpu.make_async_copy(src_hbm.at[pl.ds(0, AB)], ref, sem).wait()
        pltpu.make_async_copy(al3d_hbm.at[0], alb.at[p], sem).wait()

    def prep_and_gather(p, cch):
        for b in range(AB // 16):
            sv = srcb[p, pl.ds(b * 16, 16)]
            for h in range(4):
                idx2[p, h, pl.ds(b * 16, 16)] = sv * 16 + (h * 4 + cch)
        return [pltpu.async_copy(hf_hbm.at[idx2.at[p, h]],
                                 gbuf.at[p, pl.ds(h * AB, AB)], semg[p])
                for h in range(4)]

    def wait_gathers(p):
        for h in range(4):
            pltpu.make_async_copy(hf_hbm.at[idx2.at[p, h]],
                                  gbuf.at[p, pl.ds(h * AB, AB)], semg[p]).wait()

    def wait_scatter(p):
        pltpu.make_async_copy(msgb.at[p], spout.at[sdst.at[p]], semsc[p]).wait()

    def compute_scatter(p):
        def edge(j, ecarry):
            jb = jnp.full((16,), 0, jnp.int32) + j + half
            h0 = jnp.full((16,), 0, jnp.int32)
            a0 = plsc.load_gather(alb.at[p], [h0, jb])
            a1 = plsc.load_gather(alb.at[p], [h0 + 1, jb])
            a2 = plsc.load_gather(alb.at[p], [h0 + 2, jb])
            a3 = plsc.load_gather(alb.at[p], [h0 + 3, jb])
            for r in range(8):
                acc = (a0 * gbuf[p, j, pl.ds(r * 16, 16)]
                       + a1 * gbuf[p, AB + j, pl.ds(r * 16, 16)]
                       + a2 * gbuf[p, 2 * AB + j, pl.ds(r * 16, 16)]
                       + a3 * gbuf[p, 3 * AB + j, pl.ds(r * 16, 16)])
                msgb[p, j, pl.ds(r * 16, 16)] = acc
            return ecarry

        lax.fori_loop(0, AB, edge, None)
        for b in range(AB // 16):
            sdst[p, pl.ds(b * 16, 16)] = dstb[p, pl.ds(b * 16, 16)]
        pltpu.async_copy(msgb.at[p], spout.at[sdst.at[p]], semsc[p], add=True)

    for cc in range(2):
        cch = cid * 2 + cc
        pltpu.sync_copy(zout_hbm, msgb.at[0])
        for z in range(ROWS_T // AB):
            pltpu.sync_copy(msgb.at[0], spout.at[pl.ds(r0 + z * AB, AB)])
        plsc.subcore_barrier()

        for c in fire_reads(0, 0, semp0):
            pass
        wait_reads(0, semp0)
        g0 = prep_and_gather(0, cch)
        fire_reads(1, 1, semp1)

        def pair(kk, carry):
            k0 = 2 * kk
            k1 = 2 * kk + 1

            @pl.when(k1 < KT)
            def _a():
                wait_reads(1, semp1)
                prep_and_gather(1, cch)

            wait_gathers(0)

            @pl.when(kk > 0)
            def _w0():
                wait_scatter(0)

            compute_scatter(0)

            @pl.when(k0 + 2 < KT)
            def _b():
                fire_reads(0, k0 + 2, semp0)
                wait_reads(0, semp0)
                prep_and_gather(0, cch)

            @pl.when(k1 < KT)
            def _c():
                wait_gathers(1)

                @pl.when(kk > 0)
                def _w1():
                    wait_scatter(1)

                compute_scatter(1)

            @pl.when(k1 + 2 < KT)
            def _d():
                fire_reads(1, k1 + 2, semp1)
            return carry

        lax.fori_loop(0, (KT + 1) // 2, pair, None)
        wait_scatter(0)
        wait_scatter(1)
        plsc.subcore_barrier()
        for z in range(ROWS_T // AB):
            pltpu.sync_copy(spout.at[pl.ds(r0 + z * AB, AB)], msgb.at[0])
            pltpu.sync_copy(msgb.at[0], agg_hbm.at[cch, pl.ds(r0 + z * AB, AB)])
        plsc.subcore_barrier()


sc_agg = functools.partial(
    pl.kernel, mesh=_SC_MESH,
    compiler_params=pltpu.CompilerParams(needs_layout_passes=False),
    out_type=jax.ShapeDtypeStruct((4, NP, 128), F32),
    scratch_types=[
        pltpu.VMEM((2, AB), jnp.int32),
        pltpu.VMEM((2, AB), jnp.int32),
        pltpu.VMEM((2, AB), jnp.int32),
        pltpu.VMEM((2, 4, EB), F32),
        pltpu.VMEM((2, 4, AB), jnp.int32),
        pltpu.VMEM((2, 4 * AB, 128), F32),
        pltpu.VMEM((2, AB, 128), F32),
        pltpu.VMEM_SHARED((NP, 128), F32),
        pltpu.SemaphoreType.DMA,
        pltpu.SemaphoreType.DMA,
        pltpu.SemaphoreType.DMA,
        pltpu.SemaphoreType.DMA,
        pltpu.SemaphoreType.DMA,
        pltpu.SemaphoreType.DMA,
    ],
)(_sc_agg_body)


def _edges_sc(src, dst, aa, gmax8, wself8, h):
    gm16 = jnp.pad(gmax8.reshape(-1), (0, 8))
    z128 = jnp.zeros((128, 128), F32)
    w3d, den2 = sc_edgew(src, dst, aa, gm16, z128)
    invp = invden(den2, wself8)
    al3d = sc_alpha(dst, w3d, invp)
    hf = h.reshape(N * 16, 128)
    zout = jnp.zeros((AB, 128), F32)
    agg3 = sc_agg(src, dst, al3d, hf, zout)
    return agg3, invp


def _gat_layer(xin, src, dst, W, asad, bias, gamma, beta, batch3):
    h, aa = mm_aa(xin, W, asad)
    gmax8, wself8 = prep(aa)
    agg3, invp = _edges_sc(src, dst, aa, gmax8, wself8, h)
    gat, stats = post(agg3, h, invp, bias.reshape(1, DH))
    y, p = bnpool(gat, stats, gamma.reshape(1, DH), beta.reshape(1, DH), batch3)
    return y, p


def _make_asad(a_s, a_d):
    idx = jnp.arange(H * DH)
    hid = idx // DH
    As = jnp.zeros((H * DH, H), F32).at[idx, hid].set(a_s.reshape(-1))
    Ad = jnp.zeros((H * DH, H), F32).at[idx, hid].set(a_d.reshape(-1))
    return jnp.concatenate([As, Ad, jnp.zeros((H * DH, 120), F32)], axis=1)


def kernel(x, edge_index, batch, W1, as1, ad1, b1, g1, be1, W2, as2, ad2, b2,
           g2, be2, W3, as3, ad3, b3, g3, be3, fw1, fb1, fw2, fb2):
    src = edge_index[0]
    dst = edge_index[1]
    batch3 = batch.reshape(NB, 1, RB)
    y1, p1 = _gat_layer(x, src, dst, W1, _make_asad(as1, ad1), b1, g1, be1, batch3)
    y2, p2 = _gat_layer(y1, src, dst, W2, _make_asad(as2, ad2), b2, g2, be2, batch3)
    y3, p3 = _gat_layer(y2, src, dst, W3, _make_asad(as3, ad3), b3, g3, be3, batch3)
    return mlp(p1, p2, p3, fw1, fb1, fw2, fb2)
